# baseline (speedup 1.0000x reference)
import jax
import jax.numpy as jnp
from jax.experimental import pallas as pl
from jax.experimental.pallas import tpu as pltpu

_NC = 100
_B = 65536
_R = 1024
_G = _B // _R


def _body(x_ref, t_ref, w_ref, out_ref, acc_ref):
    i = pl.program_id(0)

    @pl.when(i == 0)
    def _():
        acc_ref[...] = jnp.zeros_like(acc_ref)

    x = x_ref[...]
    t = t_ref[...]
    w = w_ref[...]

    m = jnp.max(x, axis=1, keepdims=True)
    e = jnp.exp(x - m) * w
    s = jnp.sum(e, axis=1, keepdims=True)
    cls = jax.lax.broadcasted_iota(jnp.int32, (_R, _NC), 1)
    onehot = (t == cls).astype(jnp.float32)
    et = jnp.sum(e * onehot, axis=1, keepdims=True)
    p = jnp.clip(et / s, 1e-5, 1.0)

    cls128 = jax.lax.broadcasted_iota(jnp.int32, (_R, 128), 1)
    oh128 = (t == cls128).astype(jnp.float32)
    acc_ref[0:1, :] += jnp.sum(p * oh128, axis=0, keepdims=True)
    acc_ref[1:2, :] += jnp.sum(oh128, axis=0, keepdims=True)

    @pl.when(i == _G - 1)
    def _():
        sums = acc_ref[0:1, :]
        counts = acc_ref[1:2, :]
        exist = counts != 0.0
        denom = jnp.where(exist, counts, 1.0)
        meanp = sums / denom
        safe = jnp.where(exist, meanp, 1.0)
        ml = -jnp.log(safe)
        pw = jnp.where(exist, ml * ml * ml, 0.0)
        n_exist = jnp.sum(exist.astype(jnp.float32))
        msum = jnp.sum(pw) / n_exist
        loss = jnp.exp(jnp.log(msum) / 3.0)
        out_ref[...] = jnp.broadcast_to(loss, (1, 1))


def kernel(output, target, weight):
    res = pl.pallas_call(
        _body,
        grid=(_G,),
        in_specs=[
            pl.BlockSpec((_R, _NC), lambda i: (i, 0)),
            pl.BlockSpec((_R, 1), lambda i: (i, 0)),
            pl.BlockSpec((1, _NC), lambda i: (0, 0)),
        ],
        out_specs=pl.BlockSpec((1, 1), lambda i: (0, 0)),
        out_shape=jax.ShapeDtypeStruct((1, 1), jnp.float32),
        scratch_shapes=[pltpu.VMEM((2, 128), jnp.float32)],
        compiler_params=pltpu.CompilerParams(
            dimension_semantics=("arbitrary",)),
    )(output, target.reshape(_B, 1), weight.reshape(1, _NC))
    return res[0, 0]


# speedup vs baseline: 1.0601x; 1.0601x over previous
import jax
import jax.numpy as jnp
from jax.experimental import pallas as pl
from jax.experimental.pallas import tpu as pltpu

_NC = 100
_B = 65536
_R = 1024
_G = _B // _R


def _body(x_ref, t_ref, w_ref, out_ref, acc_ref):
    i = pl.program_id(0)

    @pl.when(i == 0)
    def _():
        acc_ref[...] = jnp.zeros_like(acc_ref)

    x = x_ref[...]
    t = t_ref[...]
    w = w_ref[...]

    e = jnp.exp(x) * w
    ones_col = jnp.ones((_NC, 1), jnp.float32)
    s = jax.lax.dot_general(e, ones_col, (((1,), (0,)), ((), ())),
                            preferred_element_type=jnp.float32)
    cls = jax.lax.broadcasted_iota(jnp.int32, (_R, _NC), 1)
    onehot = (t == cls).astype(jnp.float32)
    et = jax.lax.dot_general(e * onehot, ones_col, (((1,), (0,)), ((), ())),
                             preferred_element_type=jnp.float32)
    p = jnp.clip(et / s, 1e-5, 1.0)

    cls128 = jax.lax.broadcasted_iota(jnp.int32, (_R, 128), 1)
    oh128 = (t == cls128).astype(jnp.float32)
    pstack = jnp.concatenate([p, jnp.ones_like(p)], axis=1)
    part = jax.lax.dot_general(pstack, oh128, (((0,), (0,)), ((), ())),
                               preferred_element_type=jnp.float32)
    acc_ref[...] += part

    @pl.when(i == _G - 1)
    def _():
        sums = acc_ref[0:1, :]
        counts = acc_ref[1:2, :]
        exist = counts != 0.0
        denom = jnp.where(exist, counts, 1.0)
        meanp = sums / denom
        safe = jnp.where(exist, meanp, 1.0)
        ml = -jnp.log(safe)
        pw = jnp.where(exist, ml * ml * ml, 0.0)
        n_exist = jnp.sum(exist.astype(jnp.float32))
        msum = jnp.sum(pw) / n_exist
        loss = jnp.exp(jnp.log(msum) / 3.0)
        out_ref[...] = jnp.broadcast_to(loss, (1, 1))


def kernel(output, target, weight):
    res = pl.pallas_call(
        _body,
        grid=(_G,),
        in_specs=[
            pl.BlockSpec((_R, _NC), lambda i: (i, 0)),
            pl.BlockSpec((_R, 1), lambda i: (i, 0)),
            pl.BlockSpec((1, _NC), lambda i: (0, 0)),
        ],
        out_specs=pl.BlockSpec((1, 1), lambda i: (0, 0)),
        out_shape=jax.ShapeDtypeStruct((1, 1), jnp.float32),
        scratch_shapes=[pltpu.VMEM((2, 128), jnp.float32)],
        compiler_params=pltpu.CompilerParams(
            dimension_semantics=("arbitrary",)),
    )(output, target.reshape(_B, 1), weight.reshape(1, _NC))
    return res[0, 0]


# speedup vs baseline: 1.4528x; 1.3704x over previous
import jax
import jax.numpy as jnp
from jax.experimental import pallas as pl
from jax.experimental.pallas import tpu as pltpu

_NC = 100
_B = 65536
_R = 4096
_G = _B // _R


def _body(x_ref, t_ref, w_ref, out_ref, acc_ref):
    i = pl.program_id(0)

    @pl.when(i == 0)
    def _():
        acc_ref[...] = jnp.zeros_like(acc_ref)

    x = x_ref[...]
    t = t_ref[...]
    w = w_ref[...]

    e = jnp.exp(x) * w
    ones_col = jnp.ones((_NC, 1), jnp.float32)
    s = jax.lax.dot_general(e, ones_col, (((1,), (0,)), ((), ())),
                            preferred_element_type=jnp.float32)
    cls = jax.lax.broadcasted_iota(jnp.int32, (_R, _NC), 1)
    onehot = (t == cls).astype(jnp.float32)
    et = jax.lax.dot_general(e * onehot, ones_col, (((1,), (0,)), ((), ())),
                             preferred_element_type=jnp.float32)
    p = jnp.clip(et / s, 1e-5, 1.0)

    cls128 = jax.lax.broadcasted_iota(jnp.int32, (_R, 128), 1)
    oh128 = (t == cls128).astype(jnp.float32)
    pstack = jnp.concatenate([p, jnp.ones_like(p)], axis=1)
    part = jax.lax.dot_general(pstack, oh128, (((0,), (0,)), ((), ())),
                               preferred_element_type=jnp.float32)
    acc_ref[...] += part

    @pl.when(i == _G - 1)
    def _():
        sums = acc_ref[0:1, :]
        counts = acc_ref[1:2, :]
        exist = counts != 0.0
        denom = jnp.where(exist, counts, 1.0)
        meanp = sums / denom
        safe = jnp.where(exist, meanp, 1.0)
        ml = -jnp.log(safe)
        pw = jnp.where(exist, ml * ml * ml, 0.0)
        n_exist = jnp.sum(exist.astype(jnp.float32))
        msum = jnp.sum(pw) / n_exist
        loss = jnp.exp(jnp.log(msum) / 3.0)
        out_ref[...] = jnp.broadcast_to(loss, (1, 1))


def kernel(output, target, weight):
    res = pl.pallas_call(
        _body,
        grid=(_G,),
        in_specs=[
            pl.BlockSpec((_R, _NC), lambda i: (i, 0)),
            pl.BlockSpec((_R, 1), lambda i: (i, 0)),
            pl.BlockSpec((1, _NC), lambda i: (0, 0)),
        ],
        out_specs=pl.BlockSpec((1, 1), lambda i: (0, 0)),
        out_shape=jax.ShapeDtypeStruct((1, 1), jnp.float32),
        scratch_shapes=[pltpu.VMEM((2, 128), jnp.float32)],
        compiler_params=pltpu.CompilerParams(
            dimension_semantics=("arbitrary",)),
    )(output, target.reshape(_B, 1), weight.reshape(1, _NC))
    return res[0, 0]


# speedup vs baseline: 1.5293x; 1.0527x over previous
import jax
import jax.numpy as jnp
from jax.experimental import pallas as pl
from jax.experimental.pallas import tpu as pltpu

_NC = 100
_B = 65536
_R = 8192
_G = _B // _R


def _body(x_ref, t_ref, w_ref, out_ref, acc_ref):
    i = pl.program_id(0)

    @pl.when(i == 0)
    def _():
        acc_ref[...] = jnp.zeros_like(acc_ref)

    x = x_ref[...]
    t = t_ref[...]
    w = w_ref[...]

    e = jnp.exp(x) * w
    ones_col = jnp.ones((_NC, 1), jnp.float32)
    s = jax.lax.dot_general(e, ones_col, (((1,), (0,)), ((), ())),
                            preferred_element_type=jnp.float32)
    cls = jax.lax.broadcasted_iota(jnp.int32, (_R, _NC), 1)
    onehot = (t == cls).astype(jnp.float32)
    et = jax.lax.dot_general(e * onehot, ones_col, (((1,), (0,)), ((), ())),
                             preferred_element_type=jnp.float32)
    p = jnp.clip(et / s, 1e-5, 1.0)

    cls128 = jax.lax.broadcasted_iota(jnp.int32, (_R, 128), 1)
    oh128 = (t == cls128).astype(jnp.float32)
    pstack = jnp.concatenate([p, jnp.ones_like(p)], axis=1)
    part = jax.lax.dot_general(pstack, oh128, (((0,), (0,)), ((), ())),
                               preferred_element_type=jnp.float32)
    acc_ref[...] += part

    @pl.when(i == _G - 1)
    def _():
        sums = acc_ref[0:1, :]
        counts = acc_ref[1:2, :]
        exist = counts != 0.0
        denom = jnp.where(exist, counts, 1.0)
        meanp = sums / denom
        safe = jnp.where(exist, meanp, 1.0)
        ml = -jnp.log(safe)
        pw = jnp.where(exist, ml * ml * ml, 0.0)
        n_exist = jnp.sum(exist.astype(jnp.float32))
        msum = jnp.sum(pw) / n_exist
        loss = jnp.exp(jnp.log(msum) / 3.0)
        out_ref[...] = jnp.broadcast_to(loss, (1, 1))


def kernel(output, target, weight):
    res = pl.pallas_call(
        _body,
        grid=(_G,),
        in_specs=[
            pl.BlockSpec((_R, _NC), lambda i: (i, 0)),
            pl.BlockSpec((_R, 1), lambda i: (i, 0)),
            pl.BlockSpec((1, _NC), lambda i: (0, 0)),
        ],
        out_specs=pl.BlockSpec((1, 1), lambda i: (0, 0)),
        out_shape=jax.ShapeDtypeStruct((1, 1), jnp.float32),
        scratch_shapes=[pltpu.VMEM((2, 128), jnp.float32)],
        compiler_params=pltpu.CompilerParams(
            dimension_semantics=("arbitrary",)),
    )(output, target.reshape(_B, 1), weight.reshape(1, _NC))
    return res[0, 0]


# speedup vs baseline: 1.6095x; 1.0525x over previous
import jax
import jax.numpy as jnp
from jax.experimental import pallas as pl
from jax.experimental.pallas import tpu as pltpu

_NC = 100
_B = 65536
_R = 8192
_G = _B // _R


def _body(x_ref, t_ref, w_ref, out_ref, acc_ref):
    i = pl.program_id(0)

    @pl.when(i == 0)
    def _():
        acc_ref[...] = jnp.zeros_like(acc_ref)

    x = x_ref[...]
    t = t_ref[...]
    w = w_ref[...]

    e = jnp.exp(x) * w
    cls = jax.lax.broadcasted_iota(jnp.int32, (_R, _NC), 1)
    e_masked = jnp.where(t == cls, e, 0.0)
    ones_row = jnp.ones((1, _NC), jnp.float32)
    s = jax.lax.dot_general(ones_row, e, (((1,), (1,)), ((), ())),
                            preferred_element_type=jnp.float32)
    et = jax.lax.dot_general(ones_row, e_masked, (((1,), (1,)), ((), ())),
                             preferred_element_type=jnp.float32)
    p = jnp.clip(et / s, 1e-5, 1.0)

    cls128 = jax.lax.broadcasted_iota(jnp.int32, (_R, 128), 1)
    oh128 = (t == cls128).astype(jnp.float32)
    pstack = jnp.concatenate([p, jnp.ones_like(p)], axis=0)
    part = jax.lax.dot_general(pstack, oh128, (((1,), (0,)), ((), ())),
                               preferred_element_type=jnp.float32)
    acc_ref[...] += part

    @pl.when(i == _G - 1)
    def _():
        sums = acc_ref[0:1, :]
        counts = acc_ref[1:2, :]
        exist = counts != 0.0
        denom = jnp.where(exist, counts, 1.0)
        meanp = sums / denom
        safe = jnp.where(exist, meanp, 1.0)
        ml = -jnp.log(safe)
        pw = jnp.where(exist, ml * ml * ml, 0.0)
        n_exist = jnp.sum(exist.astype(jnp.float32))
        msum = jnp.sum(pw) / n_exist
        loss = jnp.exp(jnp.log(msum) / 3.0)
        out_ref[...] = jnp.broadcast_to(loss, (1, 1))


def kernel(output, target, weight):
    res = pl.pallas_call(
        _body,
        grid=(_G,),
        in_specs=[
            pl.BlockSpec((_R, _NC), lambda i: (i, 0)),
            pl.BlockSpec((_R, 1), lambda i: (i, 0)),
            pl.BlockSpec((1, _NC), lambda i: (0, 0)),
        ],
        out_specs=pl.BlockSpec((1, 1), lambda i: (0, 0)),
        out_shape=jax.ShapeDtypeStruct((1, 1), jnp.float32),
        scratch_shapes=[pltpu.VMEM((2, 128), jnp.float32)],
        compiler_params=pltpu.CompilerParams(
            dimension_semantics=("arbitrary",)),
    )(output, target.reshape(_B, 1), weight.reshape(1, _NC))
    return res[0, 0]
